# baseline (speedup 1.0000x reference)
import jax
import jax.numpy as jnp
from jax import lax
from jax.experimental import pallas as pl
from jax.experimental.pallas import tpu as pltpu
from jax.experimental.pallas import tpu_sc as plsc

_B, _L, _D = 4096, 200, 64
_N = _B * _L
_NC, _NS = 2, 16
_NW = _NC * _NS
_C = 128
_CHUNKS = _N // (_NW * _C)
_NBUF = 4


def _sc_body(idx_hbm, emb_hbm, tab_hbm, out_hbm, *s):
    idx_v = s[0:4]
    comb = s[4:8]
    isem = s[8:12]
    gsem = s[12:16]
    esem = s[16:20]
    psem = s[20:24]
    tab_sh = s[24]
    sid = lax.axis_index("s")
    wid = sid * _NC + lax.axis_index("c")
    wbase = wid * _CHUNKS

    @pl.when(sid == 0)
    def _():
        pltpu.sync_copy(tab_hbm, tab_sh)
    plsc.subcore_barrier()

    def issue_idx(c, r):
        pltpu.async_copy(idx_hbm.at[pl.ds(wbase + c, 1)], idx_v[r], isem[r])

    def gather(r):
        pltpu.make_async_copy(
            idx_hbm.at[pl.ds(0, 1)], idx_v[r], isem[r]).wait()
        pltpu.async_copy(tab_sh.at[idx_v[r].at[0]], comb[r], gsem[r])

    def embed_fill(c, r):
        base = (wbase + c) * _C
        pltpu.make_async_copy(tab_hbm.at[pl.ds(0, _C)], comb[r], gsem[r]).wait()
        pltpu.async_copy(emb_hbm.at[pl.ds(base, _C)],
                         comb[r].at[pl.ds(0, _C), pl.ds(0, _D)], esem[r])

    def write_out(c, r):
        base = (wbase + c) * _C
        pltpu.make_async_copy(
            emb_hbm.at[pl.ds(0, _C)],
            comb[r].at[pl.ds(0, _C), pl.ds(0, _D)], esem[r]).wait()
        pltpu.async_copy(comb[r], out_hbm.at[pl.ds(base, _C)], psem[r])

    def wait_write(r):
        pltpu.make_async_copy(comb[r], out_hbm.at[pl.ds(0, _C)], psem[r]).wait()

    def step(c, r, first=3, prefetch=True):
        gather(r)
        if first >= 1:
            embed_fill(c - 1, (r - 1) % _NBUF)
        if first >= 2:
            write_out(c - 2, (r - 2) % _NBUF)
        if first >= 3:
            wait_write((r + 1) % _NBUF)
        if prefetch:
            issue_idx(c + 2, (r + 2) % _NBUF)

    issue_idx(0, 0)
    issue_idx(1, 1)
    step(0, 0, first=0)
    step(1, 1, first=1)
    step(2, 2, first=2)
    step(3, 3)

    def loop(k, carry):
        c = 4 * k
        step(c, 0)
        step(c + 1, 1)
        step(c + 2, 2)
        step(c + 3, 3)
        return carry

    lax.fori_loop(1, _CHUNKS // 4 - 1, loop, 0)
    step(_CHUNKS - 4, 0)
    step(_CHUNKS - 3, 1)
    step(_CHUNKS - 2, 2, prefetch=False)
    step(_CHUNKS - 1, 3, prefetch=False)

    embed_fill(_CHUNKS - 1, 3)
    write_out(_CHUNKS - 2, 2)
    write_out(_CHUNKS - 1, 3)
    wait_write(1)
    wait_write(2)
    wait_write(3)


def kernel(input, embedded, pos_emb):
    idx = input.reshape(_N // 128, 128).astype(jnp.int32)
    emb = embedded.reshape(_N, _D)
    tab = jnp.pad(pos_emb, ((0, 0), (_D, 0)))
    mesh = plsc.VectorSubcoreMesh(core_axis_name="c", subcore_axis_name="s")
    out = pl.kernel(
        _sc_body,
        out_type=jax.ShapeDtypeStruct((_N, 2 * _D), jnp.float32),
        mesh=mesh,
        scratch_types=(
            [pltpu.VMEM((1, 128), jnp.int32) for _ in range(_NBUF)]
            + [pltpu.VMEM((_C, 2 * _D), jnp.float32) for _ in range(_NBUF)]
            + [pltpu.SemaphoreType.DMA for _ in range(4 * _NBUF)]
            + [pltpu.VMEM_SHARED((4096, 2 * _D), jnp.float32)]
        ),
        compiler_params=pltpu.CompilerParams(use_tc_tiling_on_sc=False),
    )(idx, emb, tab)
    return out.reshape(_B, _L, 2 * _D)
